# baseline (speedup 1.0000x reference)
import functools
import math

import jax
import jax.numpy as jnp
from jax import lax
from jax.experimental import pallas as pl
from jax.experimental.pallas import tpu as pltpu
from jax.experimental.pallas import tpu_sc as plsc

_N = 2048
_SIZE = 32000
_SMOOTHING = 0.1
_CONF = 1.0 - _SMOOTHING
_FILL = _SMOOTHING / (_SIZE - 2)
_ROW_K = _CONF * math.log(_CONF) + (_SIZE - 2) * _FILL * math.log(_FILL)

_BN = 128
_BV = 6400
_NBN = _N // _BN
_NBV = _SIZE // _BV


def _tc_body(t_ref, x_ref, out_ref, acc_ref):
    i = pl.program_id(0)
    j = pl.program_id(1)

    @pl.when(j == 0)
    def _reset():
        acc_ref[...] = jnp.zeros((_BN, 128), jnp.float32)

    x = x_ref[...]
    acc = acc_ref[...]
    for k in range(_BV // 128):
        acc += x[:, k * 128:(k + 1) * 128]
    acc_ref[...] = acc

    @pl.when((i == 0) & (j == 0))
    def _init():
        out_ref[0, 0] = 0.0

    @pl.when(j == _NBV - 1)
    def _finish():
        m = t_ref[0, 0, :][:, None] != 0
        masked = jnp.where(m, acc_ref[...], 0.0)
        out_ref[0, 0] += jnp.sum(masked)


def _tc_masked_rowsum(x, t3):
    out = pl.pallas_call(
        _tc_body,
        grid=(_NBN, _NBV),
        in_specs=[
            pl.BlockSpec((1, 1, _BN), lambda i, j: (i, 0, 0)),
            pl.BlockSpec((_BN, _BV), lambda i, j: (i, j)),
        ],
        out_specs=pl.BlockSpec((1, 1), lambda i, j: (0, 0),
                               memory_space=pltpu.SMEM),
        out_shape=jax.ShapeDtypeStruct((1, 1), jnp.float32),
        scratch_shapes=[pltpu.VMEM((_BN, 128), jnp.float32)],
        compiler_params=pltpu.CompilerParams(
            dimension_semantics=("arbitrary", "arbitrary"),
        ),
    )(t3, x)
    return out[0, 0]


_NC = 2
_NS = 16
_NW = _NC * _NS
_R = _N // _NW
_L = 16


def _sc_body(x_hbm, t_hbm, out_hbm, t_v, idx_v, vals_v, acc_v, sem):
    wid = lax.axis_index("s") * _NC + lax.axis_index("c")
    base = wid * _R
    pltpu.sync_copy(t_hbm.at[pl.ds(base, _R)], t_v)
    for k in range(_R // _L):
        tv = t_v[pl.ds(k * _L, _L)]
        rows = base + k * _L + lax.iota(jnp.int32, _L)
        idx_v[pl.ds(k * _L, _L)] = rows * _SIZE + tv
        idx_v[pl.ds(_R + k * _L, _L)] = rows * _SIZE
    pltpu.async_copy(x_hbm.at[idx_v], vals_v, sem).wait()
    acc = jnp.zeros((_L,), jnp.float32)
    for k in range(_R // _L):
        g = vals_v[pl.ds(k * _L, _L)]
        z = vals_v[pl.ds(_R + k * _L, _L)]
        tv = t_v[pl.ds(k * _L, _L)]
        term = _ROW_K - (_CONF - _FILL) * g + _FILL * z
        acc = acc + jnp.where(tv != 0, term, 0.0)
    acc_v[...] = acc
    pltpu.sync_copy(acc_v, out_hbm.at[wid])


def _sc_row_terms(x_flat, target):
    mesh = plsc.VectorSubcoreMesh(core_axis_name="c", subcore_axis_name="s")
    fn = functools.partial(
        pl.kernel,
        mesh=mesh,
        out_type=jax.ShapeDtypeStruct((_NW, _L), jnp.float32),
        scratch_types=[
            pltpu.VMEM((_R,), jnp.int32),
            pltpu.VMEM((2 * _R,), jnp.int32),
            pltpu.VMEM((2 * _R,), jnp.float32),
            pltpu.VMEM((_L,), jnp.float32),
            pltpu.SemaphoreType.DMA,
        ],
    )(_sc_body)
    return fn(x_flat, target)


def kernel(x, target):
    n, size = x.shape
    assert (n, size) == (_N, _SIZE)
    t32 = target.astype(jnp.int32)
    t3 = t32.reshape(_NBN, 1, _BN)
    a = _tc_masked_rowsum(x, t3)
    parts = _sc_row_terms(x.reshape(_N * _SIZE), t32)
    return jnp.sum(parts) - jnp.float32(_FILL) * a
